# baseline (speedup 1.0000x reference)
import jax
import jax.numpy as jnp
from jax.experimental import pallas as pl

_THRES = 0.5
_L = 4096
_BR = 1024
_NB = _L // _BR


def _rowstat_body(x_ref, a_ref):
    pi = pl.program_id(0)
    x = x_ref[...]
    col = jax.lax.broadcasted_iota(jnp.int32, (_BR, _L), 1)
    g = pi * _BR + jax.lax.broadcasted_iota(jnp.int32, (_BR, 1), 0)
    masked = jnp.where(col == g, -jnp.inf, x)
    vmax = jnp.max(masked, axis=1, keepdims=True)
    cand = jnp.where(masked == vmax, col, _L)
    inds = jnp.min(cand, axis=1, keepdims=True)
    a_ref[...] = jnp.where(vmax > _THRES, inds, -1)


def _assemble_body(a_c_ref, a_r_ref, out_ref):
    pi = pl.program_id(0)
    rowi = jax.lax.broadcasted_iota(jnp.int32, (_BR, _L), 0)
    coli = jax.lax.broadcasted_iota(jnp.int32, (_BR, _L), 1)
    g = rowi + pi * _BR
    a_i = a_c_ref[pl.ds(pi * _BR, _BR), :]
    a_j = a_r_ref[...]
    hit = (coli == g) | (coli == a_i) | (a_j == g)
    out_ref[...] = hit.astype(jnp.float32)


def kernel(input):
    x = input.reshape(_L, _L)

    a_c = pl.pallas_call(
        _rowstat_body,
        grid=(_NB,),
        in_specs=[pl.BlockSpec((_BR, _L), lambda i: (i, 0))],
        out_specs=pl.BlockSpec((_BR, 1), lambda i: (i, 0)),
        out_shape=jax.ShapeDtypeStruct((_L, 1), jnp.int32),
    )(x)

    a_r = a_c.reshape(1, _L)

    out2d = pl.pallas_call(
        _assemble_body,
        grid=(_NB,),
        in_specs=[
            pl.BlockSpec((_L, 1), lambda i: (0, 0)),
            pl.BlockSpec((1, _L), lambda i: (0, 0)),
        ],
        out_specs=pl.BlockSpec((_BR, _L), lambda i: (i, 0)),
        out_shape=jax.ShapeDtypeStruct((_L, _L), jnp.float32),
    )(a_c, a_r)

    return out2d.reshape(input.shape)


# speedup vs baseline: 1.1392x; 1.1392x over previous
import jax
import jax.numpy as jnp
from jax.experimental import pallas as pl
from jax.experimental.pallas import tpu as pltpu

_THRES = 0.5
_L = 4096
_BR = 512
_NB = _L // _BR
_BIG = _L * _L


def _fused_body(x_ref, out_ref, ac_ref, ar_ref):
    s = pl.program_id(0)

    @pl.when(s < _NB)
    def _read_phase():
        x = x_ref[...]
        col = jax.lax.broadcasted_iota(jnp.int32, (_BR, _L), 1)
        g = s * _BR + jax.lax.broadcasted_iota(jnp.int32, (_BR, 1), 0)
        masked = jnp.where(col == g, -jnp.inf, x)
        vmax = jnp.max(masked, axis=1, keepdims=True)
        cand = jnp.where(masked == vmax, col, _L)
        inds = jnp.min(cand, axis=1, keepdims=True)
        a = jnp.where(vmax > _THRES, inds, -1)
        ac_ref[pl.ds(s * _BR, _BR), :] = a
        krow = jax.lax.broadcasted_iota(jnp.int32, (_BR, _BR), 0)
        kcol = jax.lax.broadcasted_iota(jnp.int32, (_BR, _BR), 1)
        spread = jnp.where(krow == kcol, a, _BIG)
        ar_ref[0:1, pl.ds(s * _BR, _BR)] = jnp.min(spread, axis=0, keepdims=True)

    @pl.when(s >= _NB)
    def _write_phase():
        i = s - _NB
        rowi = jax.lax.broadcasted_iota(jnp.int32, (_BR, _L), 0)
        coli = jax.lax.broadcasted_iota(jnp.int32, (_BR, _L), 1)
        g = rowi + i * _BR
        a_i = ac_ref[pl.ds(i * _BR, _BR), :]
        a_j = ar_ref[...]
        hit = (coli == g) | (coli == a_i) | (a_j == g)
        out_ref[...] = hit.astype(jnp.float32)


def kernel(input):
    x = input.reshape(_L, _L)

    out2d = pl.pallas_call(
        _fused_body,
        grid=(2 * _NB,),
        in_specs=[
            pl.BlockSpec((_BR, _L), lambda s: (jnp.minimum(s, _NB - 1), 0))
        ],
        out_specs=pl.BlockSpec(
            (_BR, _L), lambda s: (jnp.maximum(s - _NB, 0), 0)
        ),
        out_shape=jax.ShapeDtypeStruct((_L, _L), jnp.float32),
        scratch_shapes=[
            pltpu.VMEM((_L, 1), jnp.int32),
            pltpu.VMEM((1, _L), jnp.int32),
        ],
    )(x)

    return out2d.reshape(input.shape)
